# baseline (speedup 1.0000x reference)
import jax
import jax.numpy as jnp
from jax.experimental import pallas as pl
from jax.experimental.pallas import tpu as pltpu

B = 2
T0, T1, T2 = 512, 512, 1024
TT = T0 + T1 + T2
N = 2048
D = 1024
XD = 1024
E = 1024
H = 16
DH = 36
TOPK = 32
NH = N // H
BT = 512

_EPS = 1e-6
_INT_MIN = -(2 ** 31)


def _layernorm_mod(xb, sc, sh):
    mu = jnp.mean(xb, axis=-1, keepdims=True)
    xc = xb - mu
    var = jnp.mean(xc * xc, axis=-1, keepdims=True)
    hn = xc * jax.lax.rsqrt(var + _EPS)
    return hn * (1.0 + sc)[None, :] + sh[None, :]


def _mod_row(emb_row, w_ref, b_ref):
    se = emb_row * jax.nn.sigmoid(emb_row)
    y = jax.lax.dot_general(se, w_ref, (((1,), (1,)), ((), ())),
                            preferred_element_type=jnp.float32)
    return y + b_ref


def _kvproj_body(xf_ref, emb_ref, mw_ref, mb_ref, kw_ref, kb_ref, vw_ref,
                 vb_ref, k_ref, v_ref, mod_ref):
    h = pl.program_id(1)

    @pl.when(h == 0)
    def _():
        mod_ref[...] = _mod_row(emb_ref[0], mw_ref[...], mb_ref[...])

    m = mod_ref[0]
    xb = xf_ref[0]
    xm = _layernorm_mod(xb, m[:XD], m[XD:])
    k = jax.lax.dot_general(xm, kw_ref[0], (((1,), (1,)), ((), ())),
                            preferred_element_type=jnp.float32)
    v = jax.lax.dot_general(xm, vw_ref[0], (((1,), (1,)), ((), ())),
                            preferred_element_type=jnp.float32)
    k_ref[0, 0] = k + kb_ref[0, 0][None, :]
    v_ref[0, 0] = v + vb_ref[0, 0][None, :]


def _head_attn(qh, kh, vh):
    lT = jax.lax.dot_general(kh, qh, (((1,), (1,)), ((), ())),
                             preferred_element_type=jnp.float32) * (1.0 / 6.0)
    m = jnp.max(lT, axis=0, keepdims=True)
    eT = jnp.exp(lT - m)

    bits = jax.lax.bitcast_convert_type(lT, jnp.int32)
    key = jnp.where(bits < 0, bits ^ 0x7FFFFFFF, bits)
    t = jnp.full((1, lT.shape[1]), _INT_MIN, jnp.int32)
    cnt = jnp.sum((key >= 0).astype(jnp.int32), axis=0, keepdims=True)
    t = jnp.where(cnt >= TOPK, jnp.zeros_like(t), t)
    for b in range(30, 9, -1):
        cand = t | (1 << b)
        cnt = jnp.sum((key >= cand).astype(jnp.int32), axis=0, keepdims=True)
        t = jnp.where(cnt >= TOPK, cand, t)

    wT = jnp.where(key >= t, eT, 0.0)
    s = jnp.sum(wT, axis=0, keepdims=True)
    wn = wT * (1.0 / s)
    return jax.lax.dot_general(wn, vh, (((0,), (0,)), ((), ())),
                               preferred_element_type=jnp.float32)


def _mega_body(x_ref, emb_ref, nw_ref, nb_ref, qw_ref, qb_ref,
               k_ref, v_ref, ow_ref, ob_ref, out_ref):
    mrow = _mod_row(emb_ref[0], nw_ref[...], nb_ref[...])
    m = mrow[0]
    xb = x_ref[0]
    xm = _layernorm_mod(xb, m[:D], m[D:])
    y = jax.lax.dot_general(xm, qw_ref[...], (((1,), (1,)), ((), ())),
                            preferred_element_type=jnp.float32)
    y = y + qb_ref[0][None, :]
    os = [_head_attn(y[:, h * DH:(h + 1) * DH], k_ref[0, h], v_ref[0, h])
          for h in range(H)]
    o = jnp.concatenate(os, axis=1)
    z = jax.lax.dot_general(o, ow_ref[...], (((1,), (1,)), ((), ())),
                            preferred_element_type=jnp.float32)
    out_ref[0] = z + ob_ref[0][None, :]


def kernel(x1, x2, x3, xf, emb, norm_W, norm_b, xfnorm_W, xfnorm_b, qW, qb, kW, kb, vW, vb, oW, ob):
    f32 = jnp.float32
    nblk = TT // BT
    emb3 = emb.reshape(B, 1, E)

    def sid(j):
        return jnp.minimum(j, 2)

    kb3 = kb.reshape(H, 1, DH)
    vb3 = vb.reshape(H, 1, DH)
    K, V = pl.pallas_call(
        _kvproj_body,
        grid=(B, H),
        in_specs=[
            pl.BlockSpec((1, NH, XD), lambda b, h: (b, h, 0)),
            pl.BlockSpec((1, 1, E), lambda b, h: (b, 0, 0)),
            pl.BlockSpec((2 * XD, E), lambda b, h: (0, 0)),
            pl.BlockSpec((1, 2 * XD), lambda b, h: (0, 0)),
            pl.BlockSpec((1, DH, XD), lambda b, h: (h, 0, 0)),
            pl.BlockSpec((1, 1, DH), lambda b, h: (h, 0, 0)),
            pl.BlockSpec((1, DH, XD), lambda b, h: (h, 0, 0)),
            pl.BlockSpec((1, 1, DH), lambda b, h: (h, 0, 0)),
        ],
        out_specs=[
            pl.BlockSpec((1, 1, NH, DH), lambda b, h: (b, h, 0, 0)),
            pl.BlockSpec((1, 1, NH, DH), lambda b, h: (b, h, 0, 0)),
        ],
        out_shape=[
            jax.ShapeDtypeStruct((B, H, NH, DH), f32),
            jax.ShapeDtypeStruct((B, H, NH, DH), f32),
        ],
        scratch_shapes=[pltpu.VMEM((1, 2 * XD), f32)],
    )(xf, emb3, xfnorm_W, xfnorm_b.reshape(1, 2 * XD), kW, kb3, vW, vb3)

    def _mega_call(i, xi):
        Ti = xi.shape[1]
        return pl.pallas_call(
            _mega_body,
            grid=(Ti // BT, B),
            in_specs=[
                pl.BlockSpec((1, BT, D), lambda j, b: (b, j, 0)),
                pl.BlockSpec((1, 1, E), lambda j, b: (b, 0, 0)),
                pl.BlockSpec((2 * D, E), lambda j, b: (0, 0)),
                pl.BlockSpec((1, 2 * D), lambda j, b: (0, 0)),
                pl.BlockSpec((H * DH, D), lambda j, b: (0, 0)),
                pl.BlockSpec((1, H * DH), lambda j, b: (0, 0)),
                pl.BlockSpec((1, H, NH, DH), lambda j, b: (b, 0, 0, 0)),
                pl.BlockSpec((1, H, NH, DH), lambda j, b: (b, 0, 0, 0)),
                pl.BlockSpec((D, H * DH), lambda j, b: (0, 0)),
                pl.BlockSpec((1, D), lambda j, b: (0, 0)),
            ],
            out_specs=pl.BlockSpec((1, BT, D), lambda j, b: (b, j, 0)),
            out_shape=jax.ShapeDtypeStruct((B, Ti, D), f32),
        )(xi, emb3, norm_W[i], norm_b[i].reshape(1, 2 * D), qW[i],
          qb[i].reshape(1, H * DH), K, V, oW[i], ob[i].reshape(1, D))

    return (_mega_call(0, x1), _mega_call(1, x2), _mega_call(2, x3))


# speedup vs baseline: 1.3333x; 1.3333x over previous
import jax
import jax.numpy as jnp
from jax.experimental import pallas as pl
from jax.experimental.pallas import tpu as pltpu

B = 2
T0, T1, T2 = 512, 512, 1024
TT = T0 + T1 + T2
N = 2048
D = 1024
XD = 1024
E = 1024
H = 16
DH = 36
TOPK = 32
NH = N // H
BT = 512

_EPS = 1e-6
_INT_MIN = -(2 ** 31)


def _layernorm_mod(xb, sc, sh):
    mu = jnp.mean(xb, axis=-1, keepdims=True)
    xc = xb - mu
    var = jnp.mean(xc * xc, axis=-1, keepdims=True)
    hn = xc * jax.lax.rsqrt(var + _EPS)
    return hn * (1.0 + sc)[None, :] + sh[None, :]


def _mod_row(emb_row, w_ref, b_ref):
    se = emb_row * jax.nn.sigmoid(emb_row)
    y = jax.lax.dot_general(se, w_ref, (((1,), (1,)), ((), ())),
                            preferred_element_type=jnp.float32)
    return y + b_ref


def _kvproj_body(xf_ref, emb_ref, mw_ref, mb_ref, kw_ref, kb_ref, vw_ref,
                 vb_ref, k_ref, v_ref, mod_ref):
    h = pl.program_id(1)

    @pl.when(h == 0)
    def _():
        mod_ref[...] = _mod_row(emb_ref[0], mw_ref[...], mb_ref[...])

    m = mod_ref[0]
    xb = xf_ref[0]
    xm = _layernorm_mod(xb, m[:XD], m[XD:])
    k = jax.lax.dot_general(xm, kw_ref[0], (((1,), (1,)), ((), ())),
                            preferred_element_type=jnp.float32)
    v = jax.lax.dot_general(xm, vw_ref[0], (((1,), (1,)), ((), ())),
                            preferred_element_type=jnp.float32)
    k_ref[0, 0] = (k + kb_ref[0, 0][None, :]) * (1.0 / 6.0)
    v_ref[0, 0] = v + vb_ref[0, 0][None, :]


def _head_attn(qh, kh, vh):
    lT = jax.lax.dot_general(kh, qh, (((1,), (1,)), ((), ())),
                             preferred_element_type=jnp.float32)
    m = jnp.max(lT, axis=0, keepdims=True)
    eT = jnp.exp(lT - m)

    key = jax.lax.bitcast_convert_type(eT, jnp.int32)
    t = jnp.zeros((1, lT.shape[1]), jnp.int32)
    for b in range(29, 10, -1):
        cand = t | (1 << b)
        cnt = jnp.sum((key >= cand).astype(jnp.int32), axis=0, keepdims=True)
        t = jnp.where(cnt >= TOPK, cand, t)

    wT = jnp.where(key >= t, eT, 0.0)
    s = jnp.sum(wT, axis=0, keepdims=True)
    wn = wT * (1.0 / s)
    return jax.lax.dot_general(wn, vh, (((0,), (0,)), ((), ())),
                               preferred_element_type=jnp.float32)


def _mega_body(x1_ref, x2_ref, x3_ref, emb_ref, nw_ref, nb_ref, qw_ref, qb_ref,
               k_ref, v_ref, ow_ref, ob_ref, out_ref):
    j = pl.program_id(0)
    sidv = jnp.minimum(j, 2)
    mrow = _mod_row(emb_ref[0], nw_ref[0], nb_ref[0])
    m = mrow[0]
    xb = jnp.where(sidv == 0, x1_ref[0],
                   jnp.where(sidv == 1, x2_ref[0], x3_ref[0]))
    xm = _layernorm_mod(xb, m[:D], m[D:])
    y = jax.lax.dot_general(xm, qw_ref[0], (((1,), (1,)), ((), ())),
                            preferred_element_type=jnp.float32)
    y = y + qb_ref[0, 0][None, :]
    os = [_head_attn(y[:, h * DH:(h + 1) * DH], k_ref[0, h], v_ref[0, h])
          for h in range(H)]
    o = jnp.concatenate(os, axis=1)
    z = jax.lax.dot_general(o, ow_ref[0], (((1,), (1,)), ((), ())),
                            preferred_element_type=jnp.float32)
    out_ref[0] = z + ob_ref[0, 0][None, :]


def kernel(x1, x2, x3, xf, emb, norm_W, norm_b, xfnorm_W, xfnorm_b, qW, qb, kW, kb, vW, vb, oW, ob):
    f32 = jnp.float32
    nblk = TT // BT
    emb3 = emb.reshape(B, 1, E)

    def sid(j):
        return jnp.minimum(j, 2)

    kb3 = kb.reshape(H, 1, DH)
    vb3 = vb.reshape(H, 1, DH)
    K, V = pl.pallas_call(
        _kvproj_body,
        grid=(B, H),
        in_specs=[
            pl.BlockSpec((1, NH, XD), lambda b, h: (b, h, 0)),
            pl.BlockSpec((1, 1, E), lambda b, h: (b, 0, 0)),
            pl.BlockSpec((2 * XD, E), lambda b, h: (0, 0)),
            pl.BlockSpec((1, 2 * XD), lambda b, h: (0, 0)),
            pl.BlockSpec((1, DH, XD), lambda b, h: (h, 0, 0)),
            pl.BlockSpec((1, 1, DH), lambda b, h: (h, 0, 0)),
            pl.BlockSpec((1, DH, XD), lambda b, h: (h, 0, 0)),
            pl.BlockSpec((1, 1, DH), lambda b, h: (h, 0, 0)),
        ],
        out_specs=[
            pl.BlockSpec((1, 1, NH, DH), lambda b, h: (b, h, 0, 0)),
            pl.BlockSpec((1, 1, NH, DH), lambda b, h: (b, h, 0, 0)),
        ],
        out_shape=[
            jax.ShapeDtypeStruct((B, H, NH, DH), f32),
            jax.ShapeDtypeStruct((B, H, NH, DH), f32),
        ],
        scratch_shapes=[pltpu.VMEM((1, 2 * XD), f32)],
    )(xf, emb3, xfnorm_W, xfnorm_b.reshape(1, 2 * XD), kW, kb3, vW, vb3)

    qb3 = qb.reshape(3, 1, H * DH)
    ob3 = ob.reshape(3, 1, D)
    nb3 = norm_b.reshape(3, 1, 2 * D)
    y = pl.pallas_call(
        _mega_body,
        grid=(nblk, B),
        in_specs=[
            pl.BlockSpec((1, BT, D), lambda j, b: (b, 0, 0)),
            pl.BlockSpec((1, BT, D), lambda j, b: (b, 0, 0)),
            pl.BlockSpec((1, BT, D), lambda j, b: (b, jnp.maximum(j - 2, 0), 0)),
            pl.BlockSpec((1, 1, E), lambda j, b: (b, 0, 0)),
            pl.BlockSpec((1, 2 * D, E), lambda j, b: (sid(j), 0, 0)),
            pl.BlockSpec((1, 1, 2 * D), lambda j, b: (sid(j), 0, 0)),
            pl.BlockSpec((1, H * DH, D), lambda j, b: (sid(j), 0, 0)),
            pl.BlockSpec((1, 1, H * DH), lambda j, b: (sid(j), 0, 0)),
            pl.BlockSpec((1, H, NH, DH), lambda j, b: (b, 0, 0, 0)),
            pl.BlockSpec((1, H, NH, DH), lambda j, b: (b, 0, 0, 0)),
            pl.BlockSpec((1, D, H * DH), lambda j, b: (sid(j), 0, 0)),
            pl.BlockSpec((1, 1, D), lambda j, b: (sid(j), 0, 0)),
        ],
        out_specs=pl.BlockSpec((1, BT, D), lambda j, b: (b, j, 0)),
        out_shape=jax.ShapeDtypeStruct((B, TT, D), f32),
    )(x1, x2, x3, emb3, norm_W, nb3, qW, qb3, K, V, oW, ob3)

    return (y[:, :T0], y[:, T0:T0 + T1], y[:, T0 + T1:])
